# baseline (speedup 1.0000x reference)
import functools

import jax
import jax.numpy as jnp
from jax import lax
from jax.experimental import pallas as pl
from jax.experimental.pallas import tpu as pltpu
from jax.experimental.pallas import tpu_sc as plsc

N = 10000
E = 320000
D = 128
H = 8
DH = 16
FF = 512

PW = 64
SCOLS = PW + 16
ACOLS = 144
TCOLS = 16
BLK = 1000

NC = 2
NS = 16
NW = NC * NS
K = 72
NCH = 140
EP = NW * NCH * K
NPAD = 10240
RPT = NPAD // NS
ZR = 64


def _ln(x, a, b):
    mean = jnp.mean(x, axis=1, keepdims=True)
    xc = x - mean
    var = jnp.sum(xc * xc, axis=1, keepdims=True) * (1.0 / (D - 1))
    return a * xc / (jnp.sqrt(var) + 1e-6) + b


def _pre_body(x_ref, wh_ref, wt_ref, we_ref, ah_ref, at_ref, a_ref, b_ref,
              s_ref, t_ref):
    h = _ln(x_ref[...], a_ref[...], b_ref[...])
    dn = (((1,), (1,)), ((), ()))
    head = jnp.tanh(lax.dot_general(h, wh_ref[...], dn,
                                    preferred_element_type=jnp.float32))
    tail = jnp.tanh(lax.dot_general(h, wt_ref[...], dn,
                                    preferred_element_type=jnp.float32))
    ft = lax.dot_general(h, we_ref[...], dn,
                         preferred_element_type=jnp.float32)
    gi = lax.broadcasted_iota(jnp.int32, (D, H), 0) // DH
    gj = lax.broadcasted_iota(jnp.int32, (D, H), 1)
    g = (gi == gj).astype(jnp.float32)
    dn2 = (((1,), (0,)), ((), ()))
    eh = lax.dot_general(head * ah_ref[...], g, dn2,
                         preferred_element_type=jnp.float32)
    et = lax.dot_general(tail * at_ref[...], g, dn2,
                         preferred_element_type=jnp.float32)
    pi = lax.broadcasted_iota(jnp.int32, (D, PW), 0)
    pj = lax.broadcasted_iota(jnp.int32, (D, PW), 1)
    pe = (pi == 2 * pj).astype(jnp.float32)
    po = (pi == 2 * pj + 1).astype(jnp.float32)
    fte = lax.dot_general(ft, pe, dn2, preferred_element_type=jnp.float32)
    fto = lax.dot_general(ft, po, dn2, preferred_element_type=jnp.float32)
    ue = lax.bitcast_convert_type(fte.astype(jnp.bfloat16),
                                  jnp.uint16).astype(jnp.uint32)
    uo = lax.bitcast_convert_type(fto.astype(jnp.bfloat16),
                                  jnp.uint16).astype(jnp.uint32)
    packed = lax.bitcast_convert_type((uo << 16) | ue, jnp.float32)
    z8 = jnp.zeros((x_ref.shape[0], 8), jnp.float32)
    s_ref[...] = jnp.concatenate([packed, eh, z8], axis=1)
    t_ref[...] = jnp.concatenate([et, z8], axis=1)


def _post_body(x_ref, a0_ref, a1_ref, a_ref, b_ref, w1_ref, b1_ref,
               w2_ref, b2_ref, o_ref):
    acc = a0_ref[...] + a1_ref[...]
    featp = acc[:, 0:D]
    esum = acc[:, D:D + H]
    inv = jnp.where(esum > 0, 1.0 / esum, 0.0)
    j8 = lax.broadcasted_iota(jnp.int32, (H, D), 1)
    orig8 = 32 * ((j8 % PW) // DH) + 2 * (j8 % DH) + (j8 >= PW)
    h8 = lax.broadcasted_iota(jnp.int32, (H, D), 0)
    rp = (orig8 // DH == h8).astype(jnp.float32)
    dn2 = (((1,), (0,)), ((), ()))
    rep = lax.dot_general(inv, rp, dn2, preferred_element_type=jnp.float32)
    jj = lax.broadcasted_iota(jnp.int32, (D, D), 0)
    cc = lax.broadcasted_iota(jnp.int32, (D, D), 1)
    orig = 32 * ((jj % PW) // DH) + 2 * (jj % DH) + (jj >= PW)
    pmat = (cc == orig).astype(jnp.float32)
    feat = lax.dot_general(featp * rep, pmat, dn2,
                           preferred_element_type=jnp.float32)
    rst = x_ref[...] + feat
    y = _ln(rst, a_ref[...], b_ref[...])
    dn = (((1,), (1,)), ((), ()))
    mid = jnp.maximum(
        lax.dot_general(y, w1_ref[...], dn,
                        preferred_element_type=jnp.float32) + b1_ref[...],
        0.0)
    ffout = lax.dot_general(mid, w2_ref[...], dn,
                            preferred_element_type=jnp.float32) + b2_ref[...]
    o_ref[...] = rst + ffout


def _edge_body(s_hbm, t_hbm, src3_hbm, dst3_hbm, out_hbm,
               si0, si1, si2, si3, di0, di1, di2, di3,
               grow0, grow1, sbuf0, sbuf1, trow0, trow1, accum,
               g0, g1, sc0, sc1, i0, i1, i2, i3):
    c = lax.axis_index("c")
    s = lax.axis_index("s")
    wid = s * NC + c

    lanes = lax.iota(jnp.int32, 16)
    mskh = lanes < H
    msk8 = lanes < 8
    grow = (grow0, grow1)
    sbuf = (sbuf0, sbuf1)
    trow = (trow0, trow1)
    sibuf = (si0, si1, si2, si3)
    dibuf = (di0, di1, di2, di3)
    gsem = (g0, g1)
    ssem = (sc0, sc1)
    isem = (i0, i1, i2, i3)

    def start_idx(j, r):
        jc = jnp.minimum(j, NCH - 1)
        pltpu.async_copy(src3_hbm.at[wid, jc], sibuf[r], isem[r])
        pltpu.async_copy(dst3_hbm.at[wid, jc], dibuf[r], isem[r])

    def wait_idx(r):
        pltpu.make_async_copy(src3_hbm.at[wid, 0], sibuf[r], isem[r]).wait()
        pltpu.make_async_copy(dst3_hbm.at[wid, 0], dibuf[r], isem[r]).wait()

    def start_gather(r, b):
        pltpu.async_copy(s_hbm.at[sibuf[r]], grow[b], gsem[b])
        pltpu.async_copy(t_hbm.at[dibuf[r]], trow[b], gsem[b])

    def wait_gather(b):
        pltpu.make_async_copy(s_hbm.at[sibuf[0]], grow[b], gsem[b]).wait()
        pltpu.make_async_copy(t_hbm.at[dibuf[0]], trow[b], gsem[b]).wait()

    def start_scatter(r, b):
        pltpu.async_copy(sbuf[b], accum.at[dibuf[r]], ssem[b], add=True)

    def wait_scatter(b):
        pltpu.make_async_copy(sbuf[b], accum.at[dibuf[0]], ssem[b]).wait()

    def compute(b):
        gb = grow[b]
        ob = sbuf[b]
        tb = trow[b]

        def edge(e, carry2):
            ehv = gb[e, pl.ds(PW, 16)]
            etv = tb[e, pl.ds(0, 16)]
            xe = ehv + etv
            t = jnp.where(xe >= 0, xe, 0.2 * xe)
            ex = jnp.where(mskh, jnp.exp(t - etv), 0.0)
            ob[e, pl.ds(2 * PW, 16)] = ex
            for q in range(4):
                v = lax.bitcast_convert_type(gb[e, pl.ds(q * 16, 16)],
                                             jnp.int32)
                ve = lax.bitcast_convert_type(v << 16, jnp.float32)
                vo = lax.bitcast_convert_type(
                    v & jnp.int32(-65536), jnp.float32)
                sq = jnp.where(msk8, ex[2 * q], ex[2 * q + 1])
                ob[e, pl.ds(q * 16, 16)] = ve * sq
                ob[e, pl.ds(PW + q * 16, 16)] = vo * sq
            return carry2

        lax.fori_loop(0, K, edge, 0)

    def step(j, r, b, first):
        rn = (r + 2) % 4
        wait_gather(b)
        if not first:
            wait_scatter(b)
        start_idx(j + 2, rn)
        compute(b)
        start_scatter(r, b)
        wait_idx(rn)
        start_gather(rn, b)

    start_idx(0, 0)
    start_idx(1, 1)
    zero16 = jnp.zeros((16,), jnp.float32)

    def zrow(rr, carry):
        for j in range(ACOLS // 16):
            sbuf0[rr, pl.ds(j * 16, 16)] = zero16
        return carry

    lax.fori_loop(0, ZR, zrow, 0)
    zsrc = sbuf0.at[pl.ds(0, ZR)]
    for q in range(RPT // ZR):
        pltpu.sync_copy(zsrc, accum.at[pl.ds(s * RPT + q * ZR, ZR)])
    wait_idx(0)
    start_gather(0, 0)
    wait_idx(1)
    start_gather(1, 1)
    plsc.subcore_barrier()
    for j in range(4):
        step(jnp.int32(j), j % 4, j % 2, first=(j < 2))

    def quad(qq, carry):
        a = 4 * qq
        for u in range(4):
            step(a + u, u, u % 2, first=False)
        return carry

    lax.fori_loop(1, NCH // 4, quad, 0)
    wait_scatter(0)
    wait_scatter(1)
    wait_gather(0)
    wait_gather(1)
    plsc.subcore_barrier()
    pltpu.sync_copy(accum.at[pl.ds(s * RPT, RPT)],
                    out_hbm.at[c, pl.ds(s * RPT, RPT)])


def kernel(ent_embed, edge_index, W_head, W_tail, W_ent, attn_h, attn_t,
           ln1_a, ln1_b, ln2_a, ln2_b, ff_w1, ff_b1, ff_w2, ff_b2):
    ah = attn_h.reshape(1, D)
    at = attn_t.reshape(1, D)
    l1a = ln1_a.reshape(1, D)
    l1b = ln1_b.reshape(1, D)
    l2a = ln2_a.reshape(1, D)
    l2b = ln2_b.reshape(1, D)
    fb1 = ff_b1.reshape(1, FF)
    fb2 = ff_b2.reshape(1, D)
    padw = (EP - E) // NW
    srcw = edge_index[0].astype(jnp.int32).reshape(NW, E // NW)
    dstw = edge_index[1].astype(jnp.int32).reshape(NW, E // NW)
    ddum = jnp.broadcast_to(
        N + (lax.iota(jnp.int32, padw) % (NPAD - N)), (NW, padw))
    src3 = jnp.concatenate(
        [srcw, jnp.zeros((NW, padw), jnp.int32)], axis=1).reshape(NW, NCH, K)
    dst3 = jnp.concatenate([dstw, ddum], axis=1).reshape(NW, NCH, K)

    full = lambda shape: pl.BlockSpec(shape, lambda i: (0, 0))
    rowblk = lambda w: pl.BlockSpec((BLK, w), lambda i: (i, 0))

    s_tab, t_tab = pl.pallas_call(
        _pre_body,
        grid=(N // BLK,),
        in_specs=[rowblk(D), full((D, D)), full((D, D)), full((D, D)),
                  full((1, D)), full((1, D)), full((1, D)), full((1, D))],
        out_specs=[rowblk(SCOLS), rowblk(TCOLS)],
        out_shape=[jax.ShapeDtypeStruct((N, SCOLS), jnp.float32),
                   jax.ShapeDtypeStruct((N, TCOLS), jnp.float32)],
    )(ent_embed, W_head, W_tail, W_ent, ah, at, l1a, l1b)
    t_tab = jnp.concatenate(
        [t_tab, jnp.zeros((NPAD - N, TCOLS), jnp.float32)])

    edge_kernel = functools.partial(
        pl.kernel,
        out_type=jax.ShapeDtypeStruct((NC, NPAD, ACOLS), jnp.float32),
        mesh=plsc.VectorSubcoreMesh(core_axis_name="c", subcore_axis_name="s"),
        scratch_types=(
            [pltpu.VMEM((K,), jnp.int32)] * 8
            + [pltpu.VMEM((K, SCOLS), jnp.float32)] * 2
            + [pltpu.VMEM((K, ACOLS), jnp.float32)] * 2
            + [pltpu.VMEM((K, TCOLS), jnp.float32)] * 2
            + [pltpu.VMEM_SHARED((NPAD, ACOLS), jnp.float32)]
            + [pltpu.SemaphoreType.DMA] * 8
        ),
        compiler_params=pltpu.CompilerParams(use_tc_tiling_on_sc=False),
    )(_edge_body)
    acc = edge_kernel(s_tab, t_tab, src3, dst3)
    acc0 = acc[0, :N]
    acc1 = acc[1, :N]

    out = pl.pallas_call(
        _post_body,
        grid=(N // BLK,),
        in_specs=[rowblk(D), rowblk(ACOLS), rowblk(ACOLS),
                  full((1, D)), full((1, D)), full((FF, D)), full((1, FF)),
                  full((D, FF)), full((1, D))],
        out_specs=rowblk(D),
        out_shape=jax.ShapeDtypeStruct((N, D), jnp.float32),
    )(ent_embed, acc0, acc1, l2a, l2b, ff_w1, fb1, ff_w2, fb2)
    return out


# speedup vs baseline: 1.2348x; 1.2348x over previous
import functools

import jax
import jax.numpy as jnp
from jax import lax
from jax.experimental import pallas as pl
from jax.experimental.pallas import tpu as pltpu
from jax.experimental.pallas import tpu_sc as plsc

N = 10000
E = 320000
D = 128
H = 8
DH = 16
FF = 512

PW = 64
SCOLS = PW + 16
ACOLS = 144
TCOLS = 16
BLK = 1000

NC = 2
NS = 16
NW = NC * NS
K = 72
NCH = 140
EP = NW * NCH * K
NPAD = 10240
RPT = NPAD // NS
ZR = 64


def _ln(x, a, b):
    mean = jnp.mean(x, axis=1, keepdims=True)
    xc = x - mean
    var = jnp.sum(xc * xc, axis=1, keepdims=True) * (1.0 / (D - 1))
    return a * xc / (jnp.sqrt(var) + 1e-6) + b


def _pre_body(x_ref, wh_ref, wt_ref, we_ref, ah_ref, at_ref, a_ref, b_ref,
              s_ref, t_ref):
    h = _ln(x_ref[...], a_ref[...], b_ref[...])
    dn = (((1,), (1,)), ((), ()))
    head = jnp.tanh(lax.dot_general(h, wh_ref[...], dn,
                                    preferred_element_type=jnp.float32))
    tail = jnp.tanh(lax.dot_general(h, wt_ref[...], dn,
                                    preferred_element_type=jnp.float32))
    ft = lax.dot_general(h, we_ref[...], dn,
                         preferred_element_type=jnp.float32)
    gi = lax.broadcasted_iota(jnp.int32, (D, H), 0) // DH
    gj = lax.broadcasted_iota(jnp.int32, (D, H), 1)
    g = (gi == gj).astype(jnp.float32)
    dn2 = (((1,), (0,)), ((), ()))
    eh = lax.dot_general(head * ah_ref[...], g, dn2,
                         preferred_element_type=jnp.float32)
    et = lax.dot_general(tail * at_ref[...], g, dn2,
                         preferred_element_type=jnp.float32)
    pi = lax.broadcasted_iota(jnp.int32, (D, PW), 0)
    pj = lax.broadcasted_iota(jnp.int32, (D, PW), 1)
    pe = (pi == 2 * pj).astype(jnp.float32)
    po = (pi == 2 * pj + 1).astype(jnp.float32)
    fte = lax.dot_general(ft, pe, dn2, preferred_element_type=jnp.float32)
    fto = lax.dot_general(ft, po, dn2, preferred_element_type=jnp.float32)
    ue = lax.bitcast_convert_type(fte.astype(jnp.bfloat16),
                                  jnp.uint16).astype(jnp.uint32)
    uo = lax.bitcast_convert_type(fto.astype(jnp.bfloat16),
                                  jnp.uint16).astype(jnp.uint32)
    packed = lax.bitcast_convert_type((uo << 16) | ue, jnp.float32)
    z8 = jnp.zeros((x_ref.shape[0], 8), jnp.float32)
    s_ref[...] = jnp.concatenate([packed, eh, z8], axis=1)
    t_ref[...] = jnp.concatenate([et, z8], axis=1)


def _post_body(x_ref, a0_ref, a1_ref, a_ref, b_ref, w1_ref, b1_ref,
               w2_ref, b2_ref, o_ref):
    acc = a0_ref[...] + a1_ref[...]
    featp = acc[:, 0:D]
    esum = acc[:, D:D + H]
    inv = jnp.where(esum > 0, 1.0 / esum, 0.0)
    j8 = lax.broadcasted_iota(jnp.int32, (H, D), 1)
    orig8 = 32 * ((j8 % PW) // DH) + 2 * (j8 % DH) + (j8 >= PW)
    h8 = lax.broadcasted_iota(jnp.int32, (H, D), 0)
    rp = (orig8 // DH == h8).astype(jnp.float32)
    dn2 = (((1,), (0,)), ((), ()))
    rep = lax.dot_general(inv, rp, dn2, preferred_element_type=jnp.float32)
    jj = lax.broadcasted_iota(jnp.int32, (D, D), 0)
    cc = lax.broadcasted_iota(jnp.int32, (D, D), 1)
    orig = 32 * ((jj % PW) // DH) + 2 * (jj % DH) + (jj >= PW)
    pmat = (cc == orig).astype(jnp.float32)
    feat = lax.dot_general(featp * rep, pmat, dn2,
                           preferred_element_type=jnp.float32)
    rst = x_ref[...] + feat
    y = _ln(rst, a_ref[...], b_ref[...])
    dn = (((1,), (1,)), ((), ()))
    mid = jnp.maximum(
        lax.dot_general(y, w1_ref[...], dn,
                        preferred_element_type=jnp.float32) + b1_ref[...],
        0.0)
    ffout = lax.dot_general(mid, w2_ref[...], dn,
                            preferred_element_type=jnp.float32) + b2_ref[...]
    o_ref[...] = rst + ffout


def _edge_body(s_hbm, t_hbm, src3_hbm, dst3_hbm, out_hbm,
               si0, si1, si2, si3, di0, di1, di2, di3,
               grow0, grow1, sbuf0, sbuf1, trow0, trow1, accum,
               g0, g1, sc0, sc1, i0, i1, i2, i3):
    c = lax.axis_index("c")
    s = lax.axis_index("s")
    wid = s * NC + c

    lanes = lax.iota(jnp.int32, 16)
    mskh = lanes < H
    msk8 = lanes < 8
    grow = (grow0, grow1)
    sbuf = (sbuf0, sbuf1)
    trow = (trow0, trow1)
    sibuf = (si0, si1, si2, si3)
    dibuf = (di0, di1, di2, di3)
    gsem = (g0, g1)
    ssem = (sc0, sc1)
    isem = (i0, i1, i2, i3)

    def start_idx(j, r):
        jc = jnp.minimum(j, NCH - 1)
        pltpu.async_copy(src3_hbm.at[wid, jc], sibuf[r], isem[r])
        pltpu.async_copy(dst3_hbm.at[wid, jc], dibuf[r], isem[r])

    def wait_idx(r):
        pltpu.make_async_copy(src3_hbm.at[wid, 0], sibuf[r], isem[r]).wait()
        pltpu.make_async_copy(dst3_hbm.at[wid, 0], dibuf[r], isem[r]).wait()

    def start_gather(r, b):
        pltpu.async_copy(s_hbm.at[sibuf[r]], grow[b], gsem[b])
        pltpu.async_copy(t_hbm.at[dibuf[r]], trow[b], gsem[b])

    def wait_gather(b):
        pltpu.make_async_copy(s_hbm.at[sibuf[0]], grow[b], gsem[b]).wait()
        pltpu.make_async_copy(t_hbm.at[dibuf[0]], trow[b], gsem[b]).wait()

    def start_scatter(r, b):
        pltpu.async_copy(sbuf[b], accum.at[dibuf[r]], ssem[b], add=True)

    def wait_scatter(b):
        pltpu.make_async_copy(sbuf[b], accum.at[dibuf[0]], ssem[b]).wait()

    def compute(b):
        gb = grow[b]
        ob = sbuf[b]
        tb = trow[b]

        def edge(e, carry2):
            ehv = gb[e, pl.ds(PW, 16)]
            etv = tb[e, pl.ds(0, 16)]
            xe = ehv + etv
            t = jnp.where(xe >= 0, xe, 0.2 * xe)
            ex = jnp.where(mskh, jnp.exp(t - etv), 0.0)
            ob[e, pl.ds(2 * PW, 16)] = ex
            for q in range(4):
                v = lax.bitcast_convert_type(gb[e, pl.ds(q * 16, 16)],
                                             jnp.int32)
                ve = lax.bitcast_convert_type(v << 16, jnp.float32)
                vo = lax.bitcast_convert_type(
                    v & jnp.int32(-65536), jnp.float32)
                sq = jnp.where(msk8, ex[2 * q], ex[2 * q + 1])
                ob[e, pl.ds(q * 16, 16)] = ve * sq
                ob[e, pl.ds(PW + q * 16, 16)] = vo * sq
            return carry2

        lax.fori_loop(0, K, edge, 0)

    def step(j, r, b, first):
        rn = (r + 2) % 4
        wait_gather(b)
        if not first:
            wait_scatter(b)
        start_idx(j + 2, rn)
        compute(b)
        start_scatter(r, b)
        wait_idx(rn)
        start_gather(rn, b)

    start_idx(0, 0)
    start_idx(1, 1)
    zero16 = jnp.zeros((16,), jnp.float32)

    def zrow(rr, carry):
        for j in range(ACOLS // 16):
            sbuf0[rr, pl.ds(j * 16, 16)] = zero16
        return carry

    lax.fori_loop(0, ZR, zrow, 0)
    zsrc = sbuf0.at[pl.ds(0, ZR)]
    for q in range(RPT // ZR):
        pltpu.sync_copy(zsrc, accum.at[pl.ds(s * RPT + q * ZR, ZR)])
    wait_idx(0)
    start_gather(0, 0)
    wait_idx(1)
    start_gather(1, 1)
    plsc.subcore_barrier()
    for j in range(4):
        step(jnp.int32(j), j % 4, j % 2, first=(j < 2))

    def quad(qq, carry):
        a = 4 * qq
        for u in range(4):
            step(a + u, u, u % 2, first=False)
        return carry

    lax.fori_loop(1, NCH // 4, quad, 0)
    wait_scatter(0)
    wait_scatter(1)
    wait_gather(0)
    wait_gather(1)
    plsc.subcore_barrier()
    pltpu.sync_copy(accum.at[pl.ds(s * RPT, RPT)],
                    out_hbm.at[c, pl.ds(s * RPT, RPT)])


def kernel(ent_embed, edge_index, W_head, W_tail, W_ent, attn_h, attn_t,
           ln1_a, ln1_b, ln2_a, ln2_b, ff_w1, ff_b1, ff_w2, ff_b2):
    ah = attn_h.reshape(1, D)
    at = attn_t.reshape(1, D)
    l1a = ln1_a.reshape(1, D)
    l1b = ln1_b.reshape(1, D)
    l2a = ln2_a.reshape(1, D)
    l2b = ln2_b.reshape(1, D)
    fb1 = ff_b1.reshape(1, FF)
    fb2 = ff_b2.reshape(1, D)
    pad = EP - E
    src = jnp.concatenate(
        [edge_index[0].astype(jnp.int32), jnp.zeros((pad,), jnp.int32)])
    dst = jnp.concatenate(
        [edge_index[1].astype(jnp.int32),
         N + (lax.iota(jnp.int32, pad) % (NPAD - N))])
    src3 = src.reshape(NW, NCH, K)
    dst3 = dst.reshape(NW, NCH, K)

    full = lambda shape: pl.BlockSpec(shape, lambda i: (0, 0))
    rowblk = lambda w: pl.BlockSpec((BLK, w), lambda i: (i, 0))

    s_tab, t_tab = pl.pallas_call(
        _pre_body,
        grid=(N // BLK,),
        in_specs=[rowblk(D), full((D, D)), full((D, D)), full((D, D)),
                  full((1, D)), full((1, D)), full((1, D)), full((1, D))],
        out_specs=[rowblk(SCOLS), rowblk(TCOLS)],
        out_shape=[jax.ShapeDtypeStruct((N, SCOLS), jnp.float32),
                   jax.ShapeDtypeStruct((N, TCOLS), jnp.float32)],
    )(ent_embed, W_head, W_tail, W_ent, ah, at, l1a, l1b)
    t_tab = jnp.concatenate(
        [t_tab, jnp.zeros((NPAD - N, TCOLS), jnp.float32)])

    edge_kernel = functools.partial(
        pl.kernel,
        out_type=jax.ShapeDtypeStruct((NC, NPAD, ACOLS), jnp.float32),
        mesh=plsc.VectorSubcoreMesh(core_axis_name="c", subcore_axis_name="s"),
        scratch_types=(
            [pltpu.VMEM((K,), jnp.int32)] * 8
            + [pltpu.VMEM((K, SCOLS), jnp.float32)] * 2
            + [pltpu.VMEM((K, ACOLS), jnp.float32)] * 2
            + [pltpu.VMEM((K, TCOLS), jnp.float32)] * 2
            + [pltpu.VMEM_SHARED((NPAD, ACOLS), jnp.float32)]
            + [pltpu.SemaphoreType.DMA] * 8
        ),
        compiler_params=pltpu.CompilerParams(use_tc_tiling_on_sc=False),
    )(_edge_body)
    acc = edge_kernel(s_tab, t_tab, src3, dst3)
    acc0 = acc[0, :N]
    acc1 = acc[1, :N]

    out = pl.pallas_call(
        _post_body,
        grid=(N // BLK,),
        in_specs=[rowblk(D), rowblk(ACOLS), rowblk(ACOLS),
                  full((1, D)), full((1, D)), full((FF, D)), full((1, FF)),
                  full((D, FF)), full((1, D))],
        out_specs=rowblk(D),
        out_shape=jax.ShapeDtypeStruct((N, D), jnp.float32),
    )(ent_embed, acc0, acc1, l2a, l2b, ff_w1, fb1, ff_w2, fb2)
    return out
